# baseline (speedup 1.0000x reference)
import functools

import jax
import jax.numpy as jnp
from jax import lax
from jax.experimental import pallas as pl
from jax.experimental.pallas import tpu as pltpu
from jax.experimental.pallas import tpu_sc as plsc

B = 4096
EMBED = 64
BM = 512
NTILES = 12500
LAST_ROW = 100000
SIDE = 128
CHUNK = 32
L = 16


def _sc_gather_pack(tab, cid, subj, lvl, price, subs, revs, lect, dur):
    info = plsc.get_sparse_core_info()
    NC, NS = info.num_cores, info.num_subcores
    NW = NC * NS
    bw = B // NW
    NBUF = 4

    mesh = plsc.VectorSubcoreMesh(core_axis_name="c", subcore_axis_name="s")

    @functools.partial(
        pl.kernel,
        mesh=mesh,
        out_type=jax.ShapeDtypeStruct((B, SIDE), jnp.float32),
        scratch_types=[
            pltpu.VMEM((bw,), jnp.int32),
            pltpu.VMEM_SHARED((NW, bw), jnp.int32),
            pltpu.SMEM((bw,), jnp.int32),
            [pltpu.VMEM((8, EMBED), jnp.float32) for _ in range(NBUF)],
            pltpu.VMEM((bw, SIDE), jnp.float32),
            pltpu.VMEM((bw,), jnp.float32),
            [pltpu.SemaphoreType.DMA for _ in range(NBUF)],
        ],
        compiler_params=pltpu.CompilerParams(needs_layout_passes=False),
    )
    def k(tab_hbm, cid_hbm, subj_hbm, lvl_hbm, price_hbm, subs_hbm,
          revs_hbm, lect_hbm, dur_hbm, out_hbm,
          id_v, id_sh, id_s, bufs, out_v, scal_v, sems):
        wid = lax.axis_index("s") * NC + lax.axis_index("c")
        base = wid * bw
        iota = lax.iota(jnp.int32, L)

        pltpu.sync_copy(cid_hbm.at[pl.ds(base, bw)], id_v)
        pltpu.sync_copy(id_v, id_sh.at[wid])
        pltpu.sync_copy(id_sh.at[wid], id_s)

        def tile_of(j):
            return jnp.minimum(
                lax.shift_right_logical(id_s[j], 3), NTILES - 1)

        def start(j, b):
            pltpu.make_async_copy(
                tab_hbm.at[pl.ds(tile_of(j) * 8, 8)], bufs[b], sems[b]
            ).start()

        def extract(j, b):
            s = jnp.bitwise_and(id_s[j], 7)
            for kk in range(EMBED // L):
                out_v[j, pl.ds(kk * L, L)] = bufs[b][s, pl.ds(kk * L, L)]

        for b in range(NBUF):
            start(b, b)

        def body(g, _):
            for b in range(NBUF):
                j = g * NBUF + b
                pltpu.make_async_copy(
                    tab_hbm.at[pl.ds(0, 8)], bufs[b], sems[b]).wait()
                extract(j, b)
                @pl.when(j + NBUF < bw)
                def _():
                    start(j + NBUF, b)
            return _

        lax.fori_loop(0, bw // NBUF, body, None)

        scalars = (
            (price_hbm, 64), (subs_hbm, 65), (revs_hbm, 66), (lect_hbm, 67),
            (dur_hbm, 68),
        )
        for src, colno in scalars:
            pltpu.sync_copy(src.at[pl.ds(base, bw)], scal_v)
            for g in range(bw // L):
                vals = scal_v[pl.ds(g * L, L)]
                plsc.store_scatter(out_v, [iota + g * L, jnp.full((L,), colno, jnp.int32)], vals)
        int_cols = ((subj_hbm, 69), (lvl_hbm, 70))
        for src, colno in int_cols:
            pltpu.sync_copy(src.at[pl.ds(base, bw)], id_v)
            for g in range(bw // L):
                vals = id_v[pl.ds(g * L, L)].astype(jnp.float32)
                plsc.store_scatter(out_v, [iota + g * L, jnp.full((L,), colno, jnp.int32)], vals)
        pltpu.sync_copy(cid_hbm.at[pl.ds(base, bw)], id_v)
        for g in range(bw // L):
            vals = id_v[pl.ds(g * L, L)].astype(jnp.float32)
            plsc.store_scatter(out_v, [iota + g * L, jnp.full((L,), 71, jnp.int32)], vals)

        pltpu.sync_copy(out_v, out_hbm.at[pl.ds(base, bw)])

    return k(tab, cid, subj, lvl, price, subs, revs, lect, dur)


def _tc_body(tfidf_ref, side_ref, wt_ref, bt_ref, wn1_ref, bn1_ref,
             wn2_ref, bn2_ref, st_ref, lt_ref, wf1_ref, bf1_ref,
             wf2_ref, bf2_ref, last_ref, out_ref):
    f32 = jnp.float32
    side = side_ref[...]

    cemb = side[:, 0:EMBED]
    num = side[:, 64:69]
    subj = side[:, 69:70]
    lvl = side[:, 70:71]
    is_last = side[:, 71:72] == float(LAST_ROW)
    cemb = jnp.where(is_last, last_ref[...], cemb)

    title = jnp.maximum(
        jnp.dot(tfidf_ref[...], wt_ref[...], preferred_element_type=f32)
        + bt_ref[...], 0.0)

    h = jnp.maximum(
        jnp.dot(num, wn1_ref[...], preferred_element_type=f32)
        + bn1_ref[...], 0.0)
    nemb = jnp.maximum(
        jnp.dot(h, wn2_ref[...], preferred_element_type=f32)
        + bn2_ref[...], 0.0)

    soh = (subj == lax.broadcasted_iota(jnp.int32, (BM, 17), 1).astype(f32)
           ).astype(f32)
    semb = jnp.dot(soh, st_ref[...], preferred_element_type=f32)
    loh = (lvl == lax.broadcasted_iota(jnp.int32, (BM, 5), 1).astype(f32)
           ).astype(f32)
    lemb = jnp.dot(loh, lt_ref[...], preferred_element_type=f32)

    x1 = jnp.dot(cemb, wf1_ref[0:64, :], preferred_element_type=f32)
    x1 += jnp.dot(semb, wf1_ref[64:80, :], preferred_element_type=f32)
    x1 += jnp.dot(lemb, wf1_ref[80:88, :], preferred_element_type=f32)
    x1 += jnp.dot(title, wf1_ref[88:152, :], preferred_element_type=f32)
    x1 += jnp.dot(nemb, wf1_ref[152:160, :], preferred_element_type=f32)
    x1 = jnp.maximum(x1 + bf1_ref[...], 0.0)

    out_ref[...] = (jnp.dot(x1, wf2_ref[...], preferred_element_type=f32)
                    + bf2_ref[...])


def _tc_pipeline(tfidf, side, wt, bt, wn1, bn1, wn2, bn2, st, lt,
                 wf1, bf1, wf2, bf2, last, interpret=False):
    grid = (B // BM,)

    def batch_spec(cols):
        return pl.BlockSpec((BM, cols), lambda i: (i, 0))

    def whole(a):
        return pl.BlockSpec(a.shape, lambda i: (0,) * a.ndim)

    return pl.pallas_call(
        _tc_body,
        grid=grid,
        in_specs=[
            batch_spec(tfidf.shape[1]),
            batch_spec(SIDE),
            whole(wt), whole(bt), whole(wn1), whole(bn1), whole(wn2),
            whole(bn2), whole(st), whole(lt), whole(wf1), whole(bf1),
            whole(wf2), whole(bf2), whole(last),
        ],
        out_specs=batch_spec(EMBED),
        out_shape=jax.ShapeDtypeStruct((B, EMBED), jnp.float32),
        interpret=interpret,
    )(tfidf, side, wt, bt, wn1, bn1, wn2, bn2, st, lt, wf1, bf1, wf2, bf2,
      last)


def kernel(course_id, subject, level, title_tfidf, price, num_subscribers,
           num_reviews, num_lectures, content_duration,
           course_table, subject_table, level_table,
           W_title, b_title, W_num1, b_num1, W_num2, b_num2,
           W_f1, b_f1, W_f2, b_f2):
    last = course_table[LAST_ROW:LAST_ROW + 1]

    side = _sc_gather_pack(
        course_table, course_id.astype(jnp.int32), subject.astype(jnp.int32),
        level.astype(jnp.int32), price, num_subscribers, num_reviews,
        num_lectures, content_duration)

    return _tc_pipeline(
        title_tfidf, side,
        W_title, b_title.reshape(1, EMBED),
        W_num1, b_num1.reshape(1, -1), W_num2, b_num2.reshape(1, -1),
        subject_table, level_table, W_f1,
        b_f1.reshape(1, -1), W_f2, b_f2.reshape(1, EMBED), last)


# speedup vs baseline: 1.1523x; 1.1523x over previous
import functools

import jax
import jax.numpy as jnp
from jax import lax
from jax.experimental import pallas as pl
from jax.experimental.pallas import tpu as pltpu
from jax.experimental.pallas import tpu_sc as plsc

B = 4096
EMBED = 64
BM = 512
SIDE_T = 72
L = 16
FIRE = 16


def _sc_gather_pack(tab, cid, subj, lvl, price, subs, revs, lect, dur):
    info = plsc.get_sparse_core_info()
    NC, NS = info.num_cores, info.num_subcores
    NW = NC * NS
    bw = B // NW
    NTILES = 12500

    mesh = plsc.VectorSubcoreMesh(core_axis_name="c", subcore_axis_name="s")

    @functools.partial(
        pl.kernel,
        mesh=mesh,
        out_type=jax.ShapeDtypeStruct((SIDE_T, B), jnp.float32),
        scratch_types=[
            pltpu.VMEM((bw,), jnp.int32),
            pltpu.VMEM_SHARED((NW, bw), jnp.int32),
            pltpu.SMEM((bw,), jnp.int32),
            [pltpu.VMEM((8, EMBED), jnp.float32) for _ in range(FIRE)],
            pltpu.VMEM((SIDE_T, bw), jnp.float32),
            pltpu.VMEM((bw,), jnp.float32),
            [pltpu.SemaphoreType.DMA for _ in range(FIRE)],
        ],
        compiler_params=pltpu.CompilerParams(needs_layout_passes=False),
    )
    def k(tab_hbm, cid_hbm, subj_hbm, lvl_hbm, price_hbm, subs_hbm,
          revs_hbm, lect_hbm, dur_hbm, out_hbm,
          id_v, id_sh, id_s, bufs, out_v, scal_v, sems):
        wid = lax.axis_index("s") * NC + lax.axis_index("c")
        base = wid * bw
        iota = lax.iota(jnp.int32, L)

        pltpu.sync_copy(cid_hbm.at[pl.ds(base, bw)], id_v)
        pltpu.sync_copy(id_v, id_sh.at[wid])
        pltpu.sync_copy(id_sh.at[wid], id_s)

        def start(j, b):
            t = jnp.minimum(
                lax.shift_right_logical(id_s[j], 3), NTILES - 1)
            pltpu.make_async_copy(
                tab_hbm.at[pl.ds(t * 8, 8)], bufs[b], sems[b]).start()

        for r in range(bw // FIRE):
            for b in range(FIRE):
                start(r * FIRE + b, b)
            for b in range(FIRE):
                j = r * FIRE + b
                pltpu.make_async_copy(
                    tab_hbm.at[pl.ds(0, 8)], bufs[b], sems[b]).wait()
                s = jnp.bitwise_and(id_s[j], 7)
                for kk in range(EMBED // L):
                    vals = bufs[b][s, pl.ds(kk * L, L)]
                    plsc.store_scatter(
                        out_v, [iota + kk * L, jnp.full((L,), j, jnp.int32)],
                        vals)

        scalars = ((price_hbm, 64), (subs_hbm, 65), (revs_hbm, 66),
                   (lect_hbm, 67), (dur_hbm, 68))
        for src, row in scalars:
            pltpu.sync_copy(src.at[pl.ds(base, bw)], scal_v)
            for g in range(bw // L):
                out_v[row, pl.ds(g * L, L)] = scal_v[pl.ds(g * L, L)]
        for src, row in ((subj_hbm, 69), (lvl_hbm, 70), (cid_hbm, 71)):
            pltpu.sync_copy(src.at[pl.ds(base, bw)], id_v)
            for g in range(bw // L):
                out_v[row, pl.ds(g * L, L)] = (
                    id_v[pl.ds(g * L, L)].astype(jnp.float32))

        pltpu.sync_copy(out_v, out_hbm.at[:, pl.ds(base, bw)])

    return k(tab, cid, subj, lvl, price, subs, revs, lect, dur)


def _mm(a, b):
    return jax.lax.dot_general(
        a, b, (((1,), (0,)), ((), ())), preferred_element_type=jnp.float32)


def _tc_body(tfidfT_ref, side_ref, wtT_ref, btc_ref, wn1T_ref, bn1c_ref,
             wn2T_ref, bn2c_ref, stT_ref, ltT_ref, wf1T_ref, bf1c_ref,
             wf2T_ref, bf2c_ref, lastc_ref, out_ref):
    f32 = jnp.float32
    side = side_ref[...]

    cemb = side[0:EMBED, :]
    num = side[64:69, :]
    subj = side[69:70, :]
    lvl = side[70:71, :]
    is_last = side[71:72, :] == 100000.0
    cemb = jnp.where(is_last, lastc_ref[...], cemb)

    title = jnp.maximum(_mm(wtT_ref[...], tfidfT_ref[...]) + btc_ref[...],
                        0.0)

    h = jnp.maximum(_mm(wn1T_ref[...], num) + bn1c_ref[...], 0.0)
    nemb = jnp.maximum(_mm(wn2T_ref[...], h) + bn2c_ref[...], 0.0)

    soh = (subj == lax.broadcasted_iota(jnp.int32, (17, BM), 0).astype(f32)
           ).astype(f32)
    semb = _mm(stT_ref[...], soh)
    loh = (lvl == lax.broadcasted_iota(jnp.int32, (5, BM), 0).astype(f32)
           ).astype(f32)
    lemb = _mm(ltT_ref[...], loh)

    wf1T = wf1T_ref[...]
    x1 = _mm(wf1T[:, 0:64], cemb)
    x1 += _mm(wf1T[:, 64:80], semb)
    x1 += _mm(wf1T[:, 80:88], lemb)
    x1 += _mm(wf1T[:, 88:152], title)
    x1 += _mm(wf1T[:, 152:160], nemb)
    x1 = jnp.maximum(x1 + bf1c_ref[...], 0.0)

    out_ref[...] = _mm(wf2T_ref[...], x1) + bf2c_ref[...]


def _tc_pipeline(tfidfT, sideT, wtT, btc, wn1T, bn1c, wn2T, bn2c, stT, ltT,
                 wf1T, bf1c, wf2T, bf2c, lastc, interpret=False):
    grid = (B // BM,)

    def col_spec(rows):
        return pl.BlockSpec((rows, BM), lambda i: (0, i))

    def whole(a):
        return pl.BlockSpec(a.shape, lambda i: (0,) * a.ndim)

    return pl.pallas_call(
        _tc_body,
        grid=grid,
        in_specs=[
            col_spec(tfidfT.shape[0]),
            col_spec(SIDE_T),
            whole(wtT), whole(btc), whole(wn1T), whole(bn1c), whole(wn2T),
            whole(bn2c), whole(stT), whole(ltT), whole(wf1T), whole(bf1c),
            whole(wf2T), whole(bf2c), whole(lastc),
        ],
        out_specs=col_spec(EMBED),
        out_shape=jax.ShapeDtypeStruct((EMBED, B), jnp.float32),
        interpret=interpret,
    )(tfidfT, sideT, wtT, btc, wn1T, bn1c, wn2T, bn2c, stT, ltT, wf1T,
      bf1c, wf2T, bf2c, lastc)


def kernel(course_id, subject, level, title_tfidf, price, num_subscribers,
           num_reviews, num_lectures, content_duration,
           course_table, subject_table, level_table,
           W_title, b_title, W_num1, b_num1, W_num2, b_num2,
           W_f1, b_f1, W_f2, b_f2):
    sideT = _sc_gather_pack(
        course_table, course_id.astype(jnp.int32),
        subject.astype(jnp.int32), level.astype(jnp.int32), price,
        num_subscribers, num_reviews, num_lectures, content_duration)

    lastc = course_table[100000:100001].T

    outT = _tc_pipeline(
        title_tfidf.T, sideT,
        W_title.T, b_title.reshape(EMBED, 1),
        W_num1.T, b_num1.reshape(-1, 1),
        W_num2.T, b_num2.reshape(-1, 1),
        subject_table.T, level_table.T,
        W_f1.T, b_f1.reshape(-1, 1),
        W_f2.T, b_f2.reshape(EMBED, 1), lastc)
    return outT.T


# speedup vs baseline: 1.2522x; 1.0867x over previous
import functools

import jax
import jax.numpy as jnp
from jax import lax
from jax.experimental import pallas as pl
from jax.experimental.pallas import tpu as pltpu
from jax.experimental.pallas import tpu_sc as plsc

B = 4096
EMBED = 64
BM = 512
SIDE_T = 72
L = 16
FIRE = 16


def _sc_gather_pack(tab, cid, scal):
    info = plsc.get_sparse_core_info()
    NC, NS = info.num_cores, info.num_subcores
    NW = NC * NS
    bw = B // NW
    NTILES = 12500

    mesh = plsc.VectorSubcoreMesh(core_axis_name="c", subcore_axis_name="s")

    @functools.partial(
        pl.kernel,
        mesh=mesh,
        out_type=jax.ShapeDtypeStruct((SIDE_T, B), jnp.float32),
        scratch_types=[
            pltpu.VMEM((bw,), jnp.int32),
            pltpu.VMEM_SHARED((NW, bw), jnp.int32),
            pltpu.SMEM((bw,), jnp.int32),
            [pltpu.VMEM((8, EMBED), jnp.float32) for _ in range(FIRE)],
            pltpu.VMEM((SIDE_T, bw), jnp.float32),
            [pltpu.SemaphoreType.DMA for _ in range(FIRE)],
        ],
        compiler_params=pltpu.CompilerParams(needs_layout_passes=False),
    )
    def k(tab_hbm, cid_hbm, scal_hbm, out_hbm,
          id_v, id_sh, id_s, bufs, out_v, sems):
        wid = lax.axis_index("s") * NC + lax.axis_index("c")
        base = wid * bw
        iota = lax.iota(jnp.int32, L)

        pltpu.sync_copy(cid_hbm.at[pl.ds(base, bw)], id_v)
        pltpu.sync_copy(id_v, id_sh.at[wid])
        pltpu.sync_copy(id_sh.at[wid], id_s)

        pltpu.sync_copy(scal_hbm.at[:, pl.ds(base, bw)],
                        out_v.at[64:72, :])

        def start(j, b):
            t = jnp.minimum(
                lax.shift_right_logical(id_s[j], 3), NTILES - 1)
            pltpu.make_async_copy(
                tab_hbm.at[pl.ds(t * 8, 8)], bufs[b], sems[b]).start()

        for b in range(FIRE):
            start(b, b)

        def round_body(r, _):
            for b in range(FIRE):
                j = r * FIRE + b
                pltpu.make_async_copy(
                    tab_hbm.at[pl.ds(0, 8)], bufs[b], sems[b]).wait()
                s = jnp.bitwise_and(id_s[j], 7)
                for kk in range(EMBED // L):
                    vals = bufs[b][s, pl.ds(kk * L, L)]
                    plsc.store_scatter(
                        out_v, [iota + kk * L, jnp.full((L,), j, jnp.int32)],
                        vals)
                @pl.when(r < bw // FIRE - 1)
                def _():
                    start(j + FIRE, b)
            return _

        lax.fori_loop(0, bw // FIRE, round_body, None)

        pltpu.sync_copy(out_v, out_hbm.at[:, pl.ds(base, bw)])

    return k(tab, cid, scal)


def _mm(a, b):
    return jax.lax.dot_general(
        a, b, (((1,), (0,)), ((), ())), preferred_element_type=jnp.float32)


def _tc_body(tfidfT_ref, side_ref, wtT_ref, btc_ref, wn1T_ref, bn1c_ref,
             wn2T_ref, bn2c_ref, stT_ref, ltT_ref, wf1T_ref, bf1c_ref,
             wf2T_ref, bf2c_ref, lastc_ref, out_ref):
    f32 = jnp.float32
    side = side_ref[...]

    cemb = side[0:EMBED, :]
    num = side[64:69, :]
    subj = side[69:70, :]
    lvl = side[70:71, :]
    is_last = side[71:72, :] == 100000.0
    cemb = jnp.where(is_last, lastc_ref[...], cemb)

    title = jnp.maximum(_mm(wtT_ref[...], tfidfT_ref[...]) + btc_ref[...],
                        0.0)

    h = jnp.maximum(_mm(wn1T_ref[...], num) + bn1c_ref[...], 0.0)
    nemb = jnp.maximum(_mm(wn2T_ref[...], h) + bn2c_ref[...], 0.0)

    soh = (subj == lax.broadcasted_iota(jnp.int32, (17, BM), 0).astype(f32)
           ).astype(f32)
    semb = _mm(stT_ref[...], soh)
    loh = (lvl == lax.broadcasted_iota(jnp.int32, (5, BM), 0).astype(f32)
           ).astype(f32)
    lemb = _mm(ltT_ref[...], loh)

    wf1T = wf1T_ref[...]
    x1 = _mm(wf1T[:, 0:64], cemb)
    x1 += _mm(wf1T[:, 64:80], semb)
    x1 += _mm(wf1T[:, 80:88], lemb)
    x1 += _mm(wf1T[:, 88:152], title)
    x1 += _mm(wf1T[:, 152:160], nemb)
    x1 = jnp.maximum(x1 + bf1c_ref[...], 0.0)

    out_ref[...] = _mm(wf2T_ref[...], x1) + bf2c_ref[...]


def _tc_pipeline(tfidfT, sideT, wtT, btc, wn1T, bn1c, wn2T, bn2c, stT, ltT,
                 wf1T, bf1c, wf2T, bf2c, lastc, interpret=False):
    grid = (B // BM,)

    def col_spec(rows):
        return pl.BlockSpec((rows, BM), lambda i: (0, i))

    def whole(a):
        return pl.BlockSpec(a.shape, lambda i: (0,) * a.ndim)

    return pl.pallas_call(
        _tc_body,
        grid=grid,
        in_specs=[
            col_spec(tfidfT.shape[0]),
            col_spec(SIDE_T),
            whole(wtT), whole(btc), whole(wn1T), whole(bn1c), whole(wn2T),
            whole(bn2c), whole(stT), whole(ltT), whole(wf1T), whole(bf1c),
            whole(wf2T), whole(bf2c), whole(lastc),
        ],
        out_specs=col_spec(EMBED),
        out_shape=jax.ShapeDtypeStruct((EMBED, B), jnp.float32),
        interpret=interpret,
    )(tfidfT, sideT, wtT, btc, wn1T, bn1c, wn2T, bn2c, stT, ltT, wf1T,
      bf1c, wf2T, bf2c, lastc)


def kernel(course_id, subject, level, title_tfidf, price, num_subscribers,
           num_reviews, num_lectures, content_duration,
           course_table, subject_table, level_table,
           W_title, b_title, W_num1, b_num1, W_num2, b_num2,
           W_f1, b_f1, W_f2, b_f2):
    f32 = jnp.float32
    scal = jnp.stack([
        price, num_subscribers, num_reviews, num_lectures, content_duration,
        subject.astype(f32), level.astype(f32), course_id.astype(f32),
    ], axis=0)
    sideT = _sc_gather_pack(course_table, course_id.astype(jnp.int32), scal)

    lastc = course_table[100000:100001].T

    outT = _tc_pipeline(
        title_tfidf.T, sideT,
        W_title.T, b_title.reshape(EMBED, 1),
        W_num1.T, b_num1.reshape(-1, 1),
        W_num2.T, b_num2.reshape(-1, 1),
        subject_table.T, level_table.T,
        W_f1.T, b_f1.reshape(-1, 1),
        W_f2.T, b_f2.reshape(EMBED, 1), lastc)
    return outT.T


# speedup vs baseline: 1.6233x; 1.2963x over previous
import functools

import jax
import jax.numpy as jnp
from jax import lax
from jax.experimental import pallas as pl
from jax.experimental.pallas import tpu as pltpu
from jax.experimental.pallas import tpu_sc as plsc

B = 4096
EMBED = 64
BM = 512
L = 16
NG = 781
GSPLIT = NG * 128
CAP = 256
NBUFG = 8


def _splat(v, lane):
    idx = jnp.full((L, 1), lane, jnp.int32)
    return lax.gather(
        v, idx,
        lax.GatherDimensionNumbers(
            offset_dims=(), collapsed_slice_dims=(0,), start_index_map=(0,)),
        (1,), mode=lax.GatherScatterMode.PROMISE_IN_BOUNDS)


def _sc_gather(tabT, cid):
    info = plsc.get_sparse_core_info()
    NC, NS = info.num_cores, info.num_subcores
    NW = NC * NS
    gpw = (NG + NW - 1) // NW

    mesh = plsc.VectorSubcoreMesh(core_axis_name="c", subcore_axis_name="s")

    @functools.partial(
        pl.kernel,
        mesh=mesh,
        out_type=jax.ShapeDtypeStruct((B, 128), jnp.float32),
        scratch_types=[
            pltpu.VMEM((B,), jnp.int32),
            pltpu.VMEM((CAP,), jnp.int32),
            pltpu.VMEM((CAP // L, L), jnp.int32),
            pltpu.VMEM((NBUFG, EMBED, 128), jnp.float32),
            pltpu.VMEM((CAP, 128), jnp.float32),
            pltpu.SemaphoreType.DMA,
            pltpu.SemaphoreType.DMA,
        ],
        compiler_params=pltpu.CompilerParams(needs_layout_passes=False),
    )
    def k(tab_hbm, cid_hbm, out_hbm, id_all, mid_v, mpos_v, bufs, rows_v,
          gsem, ssem):
        wid = lax.axis_index("s") * NC + lax.axis_index("c")
        g_lo = wid * gpw
        g_hi = jnp.minimum(g_lo + gpw, NG)
        iota = lax.iota(jnp.int32, L)

        pltpu.sync_copy(cid_hbm, id_all)

        def zero_body(c, _):
            mid_v[pl.ds(c * L, L)] = jnp.zeros((L,), jnp.int32)
            return _
        lax.fori_loop(0, CAP // L, zero_body, None)

        def compact_body(i, off):
            ids = id_all[pl.ds(i * L, L)]
            g = lax.shift_right_logical(ids, 7)
            m = jnp.logical_and(g >= g_lo, g < g_hi)
            mi = m.astype(jnp.int32)
            rank = plsc.cumsum(mi)
            tot = jnp.sum(mi)
            idxs = off + rank - 1
            ok = jnp.logical_and(m, idxs < CAP)
            plsc.store_scatter(mid_v, [idxs], ids, mask=ok)
            plsc.store_scatter(
                mpos_v, [lax.shift_right_logical(idxs, 4),
                         jnp.bitwise_and(idxs, L - 1)],
                iota + i * L, mask=ok)
            return off + tot

        M = lax.fori_loop(0, B // L, compact_body, jnp.int32(0))
        M = jnp.minimum(M, CAP)
        nchunks = lax.div(M + L - 1, jnp.int32(L))

        @pl.when(M > 0)
        def _():
            lastrow = mpos_v[lax.div(M - 1, jnp.int32(L)), :]
            lastpos = _splat(lastrow, jnp.bitwise_and(M - 1, L - 1))
            plsc.store_scatter(
                mpos_v, [jnp.full((L,), nchunks - 1, jnp.int32), iota],
                lastpos, mask=(iota + (nchunks - 1) * L) >= M)

        for blk in range(gpw // NBUFG + 1):
            gc = g_lo + blk * NBUFG
            for b in range(NBUFG):
                @pl.when(gc + b < g_hi)
                def _(b=b):
                    pltpu.make_async_copy(
                        tab_hbm.at[:, pl.ds((gc + b) * 128, 128)],
                        bufs.at[b], gsem).start()
            for b in range(NBUFG):
                @pl.when(gc + b < g_hi)
                def _(b=b):
                    pltpu.make_async_copy(
                        tab_hbm.at[:, pl.ds(0, 128)], bufs.at[b],
                        gsem).wait()

            def ext_body(ci, _):
                mids = mid_v[pl.ds(ci * L, L)]
                g = lax.shift_right_logical(mids, 7)
                slot = g - gc
                inb = jnp.logical_and(
                    jnp.logical_and(g >= gc, g < jnp.minimum(gc + NBUFG, g_hi)),
                    (iota + ci * L) < M)
                col = jnp.bitwise_and(mids, 127)
                rowi = iota + ci * L
                for d in range(EMBED):
                    vals = plsc.load_gather(
                        bufs, [slot, jnp.full((L,), d, jnp.int32), col],
                        mask=inb)
                    plsc.store_scatter(
                        rows_v, [rowi, jnp.full((L,), d, jnp.int32)], vals,
                        mask=inb)
                return _

            lax.fori_loop(0, nchunks, ext_body, None)

        @pl.when(M > 0)
        def _():
            def dup_body(m, _):
                for kk in range(128 // L):
                    rows_v[m, pl.ds(kk * L, L)] = rows_v[M - 1,
                                                         pl.ds(kk * L, L)]
                return _
            lax.fori_loop(M, nchunks * L, dup_body, None)

        def scat_body(c, _):
            pltpu.make_async_copy(
                rows_v.at[pl.ds(c * L, L), :], out_hbm.at[mpos_v.at[c]],
                ssem).start()
            return _
        lax.fori_loop(0, nchunks, scat_body, None)

        def sdrain_body(c, _):
            pltpu.make_async_copy(
                rows_v.at[pl.ds(0, L), :], out_hbm.at[mpos_v.at[0]],
                ssem).wait()
            return _
        lax.fori_loop(0, nchunks, sdrain_body, None)

    return k(tabT, cid)


def _mm(a, b):
    return lax.dot_general(
        a, b, (((1,), (0,)), ((), ())), preferred_element_type=jnp.float32)


def _mm_t(a, b):
    return lax.dot_general(
        a, b, (((1,), (1,)), ((), ())), preferred_element_type=jnp.float32)


def _tc_body(tfidfT_ref, side_ref, scal_ref, wtT_ref, btc_ref, wn1T_ref,
             bn1c_ref, wn2T_ref, bn2c_ref, stT_ref, ltT_ref, wf1T_ref,
             bf1c_ref, wf2T_ref, bf2c_ref, lastT_ref, out_ref):
    f32 = jnp.float32
    scal = scal_ref[...]

    num = scal[0:5, :]
    subj = scal[5:6, :]
    lvl = scal[6:7, :]
    cid = scal[7:8, :]

    is_special = cid >= float(GSPLIT)
    spec_col = jnp.transpose(is_special.astype(f32))
    cemb_rm = side_ref[...][:, 0:EMBED]
    cemb_rm = jnp.where(spec_col > 0.5, 0.0, cemb_rm)

    title = jnp.maximum(_mm(wtT_ref[...], tfidfT_ref[...]) + btc_ref[...],
                        0.0)

    h = jnp.maximum(_mm(wn1T_ref[...], num) + bn1c_ref[...], 0.0)
    nemb = jnp.maximum(_mm(wn2T_ref[...], h) + bn2c_ref[...], 0.0)

    soh = (subj == lax.broadcasted_iota(jnp.int32, (17, BM), 0).astype(f32)
           ).astype(f32)
    semb = _mm(stT_ref[...], soh)
    loh = (lvl == lax.broadcasted_iota(jnp.int32, (5, BM), 0).astype(f32)
           ).astype(f32)
    lemb = _mm(ltT_ref[...], loh)

    ohs = (cid == (lax.broadcasted_iota(jnp.int32, (33, BM), 0)
                   .astype(f32) + float(GSPLIT))).astype(f32)
    spec_t = _mm(lastT_ref[...], ohs)

    wf1T = wf1T_ref[...]
    x1 = _mm_t(wf1T[:, 0:64], cemb_rm)
    x1 += _mm(wf1T[:, 0:64], spec_t)
    x1 += _mm(wf1T[:, 64:80], semb)
    x1 += _mm(wf1T[:, 80:88], lemb)
    x1 += _mm(wf1T[:, 88:152], title)
    x1 += _mm(wf1T[:, 152:160], nemb)
    x1 = jnp.maximum(x1 + bf1c_ref[...], 0.0)

    out_ref[...] = _mm(wf2T_ref[...], x1) + bf2c_ref[...]


def _tc_pipeline(tfidfT, side, scal, wtT, btc, wn1T, bn1c, wn2T, bn2c, stT,
                 ltT, wf1T, bf1c, wf2T, bf2c, lastT, interpret=False):
    grid = (B // BM,)

    def col_spec(rows):
        return pl.BlockSpec((rows, BM), lambda i: (0, i))

    def whole(a):
        return pl.BlockSpec(a.shape, lambda i: (0,) * a.ndim)

    return pl.pallas_call(
        _tc_body,
        grid=grid,
        in_specs=[
            col_spec(tfidfT.shape[0]),
            pl.BlockSpec((BM, 128), lambda i: (i, 0)),
            col_spec(8),
            whole(wtT), whole(btc), whole(wn1T), whole(bn1c), whole(wn2T),
            whole(bn2c), whole(stT), whole(ltT), whole(wf1T), whole(bf1c),
            whole(wf2T), whole(bf2c), whole(lastT),
        ],
        out_specs=col_spec(EMBED),
        out_shape=jax.ShapeDtypeStruct((EMBED, B), jnp.float32),
        interpret=interpret,
    )(tfidfT, side, scal, wtT, btc, wn1T, bn1c, wn2T, bn2c, stT, ltT, wf1T,
      bf1c, wf2T, bf2c, lastT)


def kernel(course_id, subject, level, title_tfidf, price, num_subscribers,
           num_reviews, num_lectures, content_duration,
           course_table, subject_table, level_table,
           W_title, b_title, W_num1, b_num1, W_num2, b_num2,
           W_f1, b_f1, W_f2, b_f2):
    f32 = jnp.float32
    side = _sc_gather(course_table.T, course_id.astype(jnp.int32))

    scal = jnp.stack([
        price, num_subscribers, num_reviews, num_lectures, content_duration,
        subject.astype(f32), level.astype(f32), course_id.astype(f32),
    ], axis=0)

    lastT = course_table[GSPLIT:].T

    outT = _tc_pipeline(
        title_tfidf.T, side, scal,
        W_title.T, b_title.reshape(EMBED, 1),
        W_num1.T, b_num1.reshape(-1, 1),
        W_num2.T, b_num2.reshape(-1, 1),
        subject_table.T, level_table.T,
        W_f1.T, b_f1.reshape(-1, 1),
        W_f2.T, b_f2.reshape(EMBED, 1), lastT)
    return outT.T


# speedup vs baseline: 1.6836x; 1.0372x over previous
import functools

import jax
import jax.numpy as jnp
from jax import lax
from jax.experimental import pallas as pl
from jax.experimental.pallas import tpu as pltpu
from jax.experimental.pallas import tpu_sc as plsc

B = 4096
EMBED = 64
BM = 512
L = 16
NG = 781
GSPLIT = NG * 128
CAP = 256
NBUFG = 8


def _splat(v, lane):
    idx = jnp.full((L, 1), lane, jnp.int32)
    return lax.gather(
        v, idx,
        lax.GatherDimensionNumbers(
            offset_dims=(), collapsed_slice_dims=(0,), start_index_map=(0,)),
        (1,), mode=lax.GatherScatterMode.PROMISE_IN_BOUNDS)


def _sc_gather(tabT, cid):
    info = plsc.get_sparse_core_info()
    NC, NS = info.num_cores, info.num_subcores
    NW = NC * NS
    gpw = (NG + NW - 1) // NW

    mesh = plsc.VectorSubcoreMesh(core_axis_name="c", subcore_axis_name="s")

    @functools.partial(
        pl.kernel,
        mesh=mesh,
        out_type=jax.ShapeDtypeStruct((B, 128), jnp.float32),
        scratch_types=[
            pltpu.VMEM((B,), jnp.int32),
            pltpu.VMEM((CAP,), jnp.int32),
            pltpu.VMEM((CAP // L, L), jnp.int32),
            pltpu.VMEM((NBUFG, EMBED, 128), jnp.float32),
            pltpu.VMEM((CAP, 128), jnp.float32),
            pltpu.SemaphoreType.DMA,
            pltpu.SemaphoreType.DMA,
        ],
        compiler_params=pltpu.CompilerParams(needs_layout_passes=False),
    )
    def k(tab_hbm, cid_hbm, out_hbm, id_all, mid_v, mpos_v, bufs, rows_v,
          gsem, ssem):
        wid = lax.axis_index("s") * NC + lax.axis_index("c")
        g_lo = wid * gpw
        g_hi = jnp.minimum(g_lo + gpw, NG)
        iota = lax.iota(jnp.int32, L)
        HALF = NBUFG // 2
        nblk = (gpw + HALF - 1) // HALF

        def fire_block(blk):
            gc = g_lo + blk * HALF
            half = (blk % 2) * HALF
            for b in range(HALF):
                @pl.when(gc + b < g_hi)
                def _(b=b):
                    pltpu.make_async_copy(
                        tab_hbm.at[:, pl.ds((gc + b) * 128, 128)],
                        bufs.at[half + b], gsem).start()

        def wait_block(blk):
            half = (blk % 2) * HALF
            gc = g_lo + blk * HALF
            for b in range(HALF):
                @pl.when(gc + b < g_hi)
                def _(b=b):
                    pltpu.make_async_copy(
                        tab_hbm.at[:, pl.ds(0, 128)], bufs.at[half + b],
                        gsem).wait()

        fire_block(0)
        pltpu.sync_copy(cid_hbm, id_all)

        def zero_body(c, _):
            mid_v[pl.ds(c * L, L)] = jnp.zeros((L,), jnp.int32)
            return _
        lax.fori_loop(0, CAP // L, zero_body, None)

        def compact_body(i, off):
            ids = id_all[pl.ds(i * L, L)]
            g = lax.shift_right_logical(ids, 7)
            m = jnp.logical_and(g >= g_lo, g < g_hi)
            mi = m.astype(jnp.int32)
            rank = plsc.cumsum(mi)
            tot = plsc.all_reduce_population_count(m)
            idxs = off + rank - 1
            ok = jnp.logical_and(m, idxs < CAP)
            plsc.store_scatter(mid_v, [idxs], ids, mask=ok)
            plsc.store_scatter(
                mpos_v, [lax.shift_right_logical(idxs, 4),
                         jnp.bitwise_and(idxs, L - 1)],
                iota + i * L, mask=ok)
            return off + tot

        off_vec = lax.fori_loop(0, B // L, compact_body,
                                jnp.zeros((L,), jnp.int32))
        M = jnp.minimum(jnp.max(off_vec), CAP)
        nchunks = lax.div(M + L - 1, jnp.int32(L))

        @pl.when(M > 0)
        def _():
            lastrow = mpos_v[lax.div(M - 1, jnp.int32(L)), :]
            lastpos = _splat(lastrow, jnp.bitwise_and(M - 1, L - 1))
            plsc.store_scatter(
                mpos_v, [jnp.full((L,), nchunks - 1, jnp.int32), iota],
                lastpos, mask=(iota + (nchunks - 1) * L) >= M)

        for blk in range(nblk):
            wait_block(blk)
            if blk + 1 < nblk:
                fire_block(blk + 1)
            gc = g_lo + blk * HALF
            half = (blk % 2) * HALF

            def ext_body(ci, _, gc=gc, half=half):
                mids = mid_v[pl.ds(ci * L, L)]
                g = lax.shift_right_logical(mids, 7)
                slot = g - gc + half
                inb = jnp.logical_and(
                    jnp.logical_and(g >= gc, g < jnp.minimum(gc + HALF, g_hi)),
                    (iota + ci * L) < M)
                col = jnp.bitwise_and(mids, 127)
                rowi = iota + ci * L
                for d in range(EMBED):
                    vals = plsc.load_gather(
                        bufs, [slot, jnp.full((L,), d, jnp.int32), col],
                        mask=inb)
                    plsc.store_scatter(
                        rows_v, [rowi, jnp.full((L,), d, jnp.int32)], vals,
                        mask=inb)
                return _

            lax.fori_loop(0, nchunks, ext_body, None)

        @pl.when(M > 0)
        def _():
            def dup_body(m, _):
                for kk in range(128 // L):
                    rows_v[m, pl.ds(kk * L, L)] = rows_v[M - 1,
                                                         pl.ds(kk * L, L)]
                return _
            lax.fori_loop(M, nchunks * L, dup_body, None)

        def scat_body(c, _):
            pltpu.make_async_copy(
                rows_v.at[pl.ds(c * L, L), :], out_hbm.at[mpos_v.at[c]],
                ssem).start()
            return _
        lax.fori_loop(0, nchunks, scat_body, None)

        def sdrain_body(c, _):
            pltpu.make_async_copy(
                rows_v.at[pl.ds(0, L), :], out_hbm.at[mpos_v.at[0]],
                ssem).wait()
            return _
        lax.fori_loop(0, nchunks, sdrain_body, None)

    return k(tabT, cid)


def _mm(a, b):
    return lax.dot_general(
        a, b, (((1,), (0,)), ((), ())), preferred_element_type=jnp.float32)


def _mm_t(a, b):
    return lax.dot_general(
        a, b, (((1,), (1,)), ((), ())), preferred_element_type=jnp.float32)


def _tc_body(tfidfT_ref, side_ref, scal_ref, wtT_ref, btc_ref, wn1T_ref,
             bn1c_ref, wn2T_ref, bn2c_ref, stT_ref, ltT_ref, wf1T_ref,
             bf1c_ref, wf2T_ref, bf2c_ref, lastT_ref, out_ref):
    f32 = jnp.float32
    scal = scal_ref[...]

    num = scal[0:5, :]
    subj = scal[5:6, :]
    lvl = scal[6:7, :]
    cid = scal[7:8, :]

    is_special = cid >= float(GSPLIT)
    spec_col = jnp.transpose(is_special.astype(f32))
    cemb_rm = side_ref[...][:, 0:EMBED]
    cemb_rm = jnp.where(spec_col > 0.5, 0.0, cemb_rm)

    title = jnp.maximum(_mm(wtT_ref[...], tfidfT_ref[...]) + btc_ref[...],
                        0.0)

    h = jnp.maximum(_mm(wn1T_ref[...], num) + bn1c_ref[...], 0.0)
    nemb = jnp.maximum(_mm(wn2T_ref[...], h) + bn2c_ref[...], 0.0)

    soh = (subj == lax.broadcasted_iota(jnp.int32, (17, BM), 0).astype(f32)
           ).astype(f32)
    semb = _mm(stT_ref[...], soh)
    loh = (lvl == lax.broadcasted_iota(jnp.int32, (5, BM), 0).astype(f32)
           ).astype(f32)
    lemb = _mm(ltT_ref[...], loh)

    ohs = (cid == (lax.broadcasted_iota(jnp.int32, (33, BM), 0)
                   .astype(f32) + float(GSPLIT))).astype(f32)
    spec_t = _mm(lastT_ref[...], ohs)

    wf1T = wf1T_ref[...]
    x1 = _mm_t(wf1T[:, 0:64], cemb_rm)
    x1 += _mm(wf1T[:, 0:64], spec_t)
    x1 += _mm(wf1T[:, 64:80], semb)
    x1 += _mm(wf1T[:, 80:88], lemb)
    x1 += _mm(wf1T[:, 88:152], title)
    x1 += _mm(wf1T[:, 152:160], nemb)
    x1 = jnp.maximum(x1 + bf1c_ref[...], 0.0)

    out_ref[...] = _mm(wf2T_ref[...], x1) + bf2c_ref[...]


def _tc_pipeline(tfidfT, side, scal, wtT, btc, wn1T, bn1c, wn2T, bn2c, stT,
                 ltT, wf1T, bf1c, wf2T, bf2c, lastT, interpret=False):
    grid = (B // BM,)

    def col_spec(rows):
        return pl.BlockSpec((rows, BM), lambda i: (0, i))

    def whole(a):
        return pl.BlockSpec(a.shape, lambda i: (0,) * a.ndim)

    return pl.pallas_call(
        _tc_body,
        grid=grid,
        in_specs=[
            col_spec(tfidfT.shape[0]),
            pl.BlockSpec((BM, 128), lambda i: (i, 0)),
            col_spec(8),
            whole(wtT), whole(btc), whole(wn1T), whole(bn1c), whole(wn2T),
            whole(bn2c), whole(stT), whole(ltT), whole(wf1T), whole(bf1c),
            whole(wf2T), whole(bf2c), whole(lastT),
        ],
        out_specs=col_spec(EMBED),
        out_shape=jax.ShapeDtypeStruct((EMBED, B), jnp.float32),
        interpret=interpret,
    )(tfidfT, side, scal, wtT, btc, wn1T, bn1c, wn2T, bn2c, stT, ltT, wf1T,
      bf1c, wf2T, bf2c, lastT)


def kernel(course_id, subject, level, title_tfidf, price, num_subscribers,
           num_reviews, num_lectures, content_duration,
           course_table, subject_table, level_table,
           W_title, b_title, W_num1, b_num1, W_num2, b_num2,
           W_f1, b_f1, W_f2, b_f2):
    f32 = jnp.float32
    side = _sc_gather(course_table.T, course_id.astype(jnp.int32))

    scal = jnp.stack([
        price, num_subscribers, num_reviews, num_lectures, content_duration,
        subject.astype(f32), level.astype(f32), course_id.astype(f32),
    ], axis=0)

    lastT = course_table[GSPLIT:].T

    outT = _tc_pipeline(
        title_tfidf.T, side, scal,
        W_title.T, b_title.reshape(EMBED, 1),
        W_num1.T, b_num1.reshape(-1, 1),
        W_num2.T, b_num2.reshape(-1, 1),
        subject_table.T, level_table.T,
        W_f1.T, b_f1.reshape(-1, 1),
        W_f2.T, b_f2.reshape(EMBED, 1), lastT)
    return outT.T


# speedup vs baseline: 1.8271x; 1.0852x over previous
import functools

import jax
import jax.numpy as jnp
from jax import lax
from jax.experimental import pallas as pl
from jax.experimental.pallas import tpu as pltpu
from jax.experimental.pallas import tpu_sc as plsc

B = 4096
EMBED = 64
BM = 512
L = 16
NG = 781
GSPLIT = NG * 128
CAP = 256
NBUFG = 10


def _splat(v, lane):
    idx = jnp.full((L, 1), lane, jnp.int32)
    return lax.gather(
        v, idx,
        lax.GatherDimensionNumbers(
            offset_dims=(), collapsed_slice_dims=(0,), start_index_map=(0,)),
        (1,), mode=lax.GatherScatterMode.PROMISE_IN_BOUNDS)


def _sc_gather(tabT, cid):
    info = plsc.get_sparse_core_info()
    NC, NS = info.num_cores, info.num_subcores
    NW = NC * NS
    gpw = (NG + NW - 1) // NW

    mesh = plsc.VectorSubcoreMesh(core_axis_name="c", subcore_axis_name="s")

    @functools.partial(
        pl.kernel,
        mesh=mesh,
        out_type=jax.ShapeDtypeStruct((B, 128), jnp.float32),
        scratch_types=[
            pltpu.VMEM((B,), jnp.int32),
            pltpu.VMEM((CAP,), jnp.int32),
            pltpu.VMEM((CAP // L, L), jnp.int32),
            pltpu.VMEM((NBUFG, EMBED, 128), jnp.float32),
            pltpu.VMEM((CAP, 128), jnp.float32),
            pltpu.SemaphoreType.DMA,
            pltpu.SemaphoreType.DMA,
        ],
        compiler_params=pltpu.CompilerParams(needs_layout_passes=False),
    )
    def k(tab_hbm, cid_hbm, out_hbm, id_all, mid_v, mpos_v, bufs, rows_v,
          gsem, ssem):
        wid = lax.axis_index("s") * NC + lax.axis_index("c")
        g_lo = wid * gpw
        g_hi = jnp.minimum(g_lo + gpw, NG)
        iota = lax.iota(jnp.int32, L)
        HALF = NBUFG // 2
        nblk = (gpw + HALF - 1) // HALF

        def fire_block(blk):
            gc = g_lo + blk * HALF
            half = (blk % 2) * HALF
            for b in range(HALF):
                @pl.when(gc + b < g_hi)
                def _(b=b):
                    pltpu.make_async_copy(
                        tab_hbm.at[:, pl.ds((gc + b) * 128, 128)],
                        bufs.at[half + b], gsem).start()

        def wait_block(blk):
            half = (blk % 2) * HALF
            gc = g_lo + blk * HALF
            for b in range(HALF):
                @pl.when(gc + b < g_hi)
                def _(b=b):
                    pltpu.make_async_copy(
                        tab_hbm.at[:, pl.ds(0, 128)], bufs.at[half + b],
                        gsem).wait()

        fire_block(0)
        pltpu.sync_copy(cid_hbm, id_all)

        def zero_body(c, _):
            mid_v[pl.ds(c * L, L)] = jnp.zeros((L,), jnp.int32)
            return _
        lax.fori_loop(0, CAP // L, zero_body, None)

        def compact_body(i, off):
            ids = id_all[pl.ds(i * L, L)]
            g = lax.shift_right_logical(ids, 7)
            m = jnp.logical_and(g >= g_lo, g < g_hi)
            mi = m.astype(jnp.int32)
            rank = plsc.cumsum(mi)
            tot = plsc.all_reduce_population_count(m)
            idxs = off + rank - 1
            ok = jnp.logical_and(m, idxs < CAP)
            plsc.store_scatter(mid_v, [idxs], ids, mask=ok)
            plsc.store_scatter(
                mpos_v, [lax.shift_right_logical(idxs, 4),
                         jnp.bitwise_and(idxs, L - 1)],
                iota + i * L, mask=ok)
            return off + tot

        off_vec = lax.fori_loop(0, B // L, compact_body,
                                jnp.zeros((L,), jnp.int32))
        M = jnp.minimum(jnp.max(off_vec), CAP)
        nchunks = lax.div(M + L - 1, jnp.int32(L))

        @pl.when(M > 0)
        def _():
            lastrow = mpos_v[lax.div(M - 1, jnp.int32(L)), :]
            lastpos = _splat(lastrow, jnp.bitwise_and(M - 1, L - 1))
            plsc.store_scatter(
                mpos_v, [jnp.full((L,), nchunks - 1, jnp.int32), iota],
                lastpos, mask=(iota + (nchunks - 1) * L) >= M)

        for blk in range(nblk):
            wait_block(blk)
            if blk + 1 < nblk:
                fire_block(blk + 1)
            gc = g_lo + blk * HALF
            half = (blk % 2) * HALF

            def ext_body(ci, _, gc=gc, half=half):
                mids = mid_v[pl.ds(ci * L, L)]
                g = lax.shift_right_logical(mids, 7)
                slot = g - gc + half
                inb = jnp.logical_and(
                    jnp.logical_and(g >= gc, g < jnp.minimum(gc + HALF, g_hi)),
                    (iota + ci * L) < M)
                col = jnp.bitwise_and(mids, 127)
                rowi = iota + ci * L
                for d in range(EMBED):
                    vals = plsc.load_gather(
                        bufs, [slot, jnp.full((L,), d, jnp.int32), col],
                        mask=inb)
                    plsc.store_scatter(
                        rows_v, [rowi, jnp.full((L,), d, jnp.int32)], vals,
                        mask=inb)
                return _

            lax.fori_loop(0, nchunks, ext_body, None)

        @pl.when(M > 0)
        def _():
            def dup_body(m, _):
                for kk in range(128 // L):
                    rows_v[m, pl.ds(kk * L, L)] = rows_v[M - 1,
                                                         pl.ds(kk * L, L)]
                return _
            lax.fori_loop(M, nchunks * L, dup_body, None)

        def scat_body(c, _):
            pltpu.make_async_copy(
                rows_v.at[pl.ds(c * L, L), :], out_hbm.at[mpos_v.at[c]],
                ssem).start()
            return _
        lax.fori_loop(0, nchunks, scat_body, None)

        def sdrain_body(c, _):
            pltpu.make_async_copy(
                rows_v.at[pl.ds(0, L), :], out_hbm.at[mpos_v.at[0]],
                ssem).wait()
            return _
        lax.fori_loop(0, nchunks, sdrain_body, None)

    return k(tabT, cid)


def _mm(a, b):
    return lax.dot_general(
        a, b, (((1,), (0,)), ((), ())), preferred_element_type=jnp.float32)


def _mm_t(a, b):
    return lax.dot_general(
        a, b, (((1,), (1,)), ((), ())), preferred_element_type=jnp.float32)


def _tc_body(tfidfT_ref, side_ref, scal_ref, wtT_ref, btc_ref, wn1T_ref,
             bn1c_ref, wn2T_ref, bn2c_ref, stT_ref, ltT_ref, wf1T_ref,
             bf1c_ref, wf2T_ref, bf2c_ref, lastT_ref, out_ref):
    f32 = jnp.float32
    scal = scal_ref[...]

    num = scal[0:5, :]
    subj = scal[5:6, :]
    lvl = scal[6:7, :]
    cid = scal[7:8, :]

    is_special = cid >= float(GSPLIT)
    spec_col = jnp.transpose(is_special.astype(f32))
    cemb_rm = side_ref[...][:, 0:EMBED]
    cemb_rm = jnp.where(spec_col > 0.5, 0.0, cemb_rm)

    title = jnp.maximum(_mm(wtT_ref[...], tfidfT_ref[...]) + btc_ref[...],
                        0.0)

    h = jnp.maximum(_mm(wn1T_ref[...], num) + bn1c_ref[...], 0.0)
    nemb = jnp.maximum(_mm(wn2T_ref[...], h) + bn2c_ref[...], 0.0)

    soh = (subj == lax.broadcasted_iota(jnp.int32, (17, BM), 0).astype(f32)
           ).astype(f32)
    semb = _mm(stT_ref[...], soh)
    loh = (lvl == lax.broadcasted_iota(jnp.int32, (5, BM), 0).astype(f32)
           ).astype(f32)
    lemb = _mm(ltT_ref[...], loh)

    ohs = (cid == (lax.broadcasted_iota(jnp.int32, (33, BM), 0)
                   .astype(f32) + float(GSPLIT))).astype(f32)
    spec_t = _mm(lastT_ref[...], ohs)

    wf1T = wf1T_ref[...]
    x1 = _mm_t(wf1T[:, 0:64], cemb_rm)
    x1 += _mm(wf1T[:, 0:64], spec_t)
    x1 += _mm(wf1T[:, 64:80], semb)
    x1 += _mm(wf1T[:, 80:88], lemb)
    x1 += _mm(wf1T[:, 88:152], title)
    x1 += _mm(wf1T[:, 152:160], nemb)
    x1 = jnp.maximum(x1 + bf1c_ref[...], 0.0)

    out_ref[...] = _mm(wf2T_ref[...], x1) + bf2c_ref[...]


def _tc_pipeline(tfidfT, side, scal, wtT, btc, wn1T, bn1c, wn2T, bn2c, stT,
                 ltT, wf1T, bf1c, wf2T, bf2c, lastT, interpret=False):
    grid = (B // BM,)

    def col_spec(rows):
        return pl.BlockSpec((rows, BM), lambda i: (0, i))

    def whole(a):
        return pl.BlockSpec(a.shape, lambda i: (0,) * a.ndim)

    return pl.pallas_call(
        _tc_body,
        grid=grid,
        in_specs=[
            col_spec(tfidfT.shape[0]),
            pl.BlockSpec((BM, 128), lambda i: (i, 0)),
            col_spec(8),
            whole(wtT), whole(btc), whole(wn1T), whole(bn1c), whole(wn2T),
            whole(bn2c), whole(stT), whole(ltT), whole(wf1T), whole(bf1c),
            whole(wf2T), whole(bf2c), whole(lastT),
        ],
        out_specs=col_spec(EMBED),
        out_shape=jax.ShapeDtypeStruct((EMBED, B), jnp.float32),
        interpret=interpret,
    )(tfidfT, side, scal, wtT, btc, wn1T, bn1c, wn2T, bn2c, stT, ltT, wf1T,
      bf1c, wf2T, bf2c, lastT)


def kernel(course_id, subject, level, title_tfidf, price, num_subscribers,
           num_reviews, num_lectures, content_duration,
           course_table, subject_table, level_table,
           W_title, b_title, W_num1, b_num1, W_num2, b_num2,
           W_f1, b_f1, W_f2, b_f2):
    f32 = jnp.float32
    side = _sc_gather(course_table.T, course_id.astype(jnp.int32))

    scal = jnp.stack([
        price, num_subscribers, num_reviews, num_lectures, content_duration,
        subject.astype(f32), level.astype(f32), course_id.astype(f32),
    ], axis=0)

    lastT = course_table[GSPLIT:].T

    outT = _tc_pipeline(
        title_tfidf.T, side, scal,
        W_title.T, b_title.reshape(EMBED, 1),
        W_num1.T, b_num1.reshape(-1, 1),
        W_num2.T, b_num2.reshape(-1, 1),
        subject_table.T, level_table.T,
        W_f1.T, b_f1.reshape(-1, 1),
        W_f2.T, b_f2.reshape(EMBED, 1), lastT)
    return outT.T
